# baseline (speedup 1.0000x reference)
import jax
import jax.numpy as jnp
from jax import lax
from jax.experimental import pallas as pl
from jax.experimental.pallas import tpu as pltpu
from jax.experimental.pallas import tpu_sc as plsc
from jax._src.pallas import mpmd as _mpmd

M = 65536
B = 16384
D = 256
NC = 2
NS = 16
L = 16
CH = 64
NCHUNK = B // CH
NROWT = 24
BASE_CHUNKS = NCHUNK // NROWT
CHD = 128
CHB = 2048
Q = 4
BQ = B // Q
UCH = 128
NU = 8
T2 = 0.85 * 0.85
NG = CH // L

_mesh = lambda: plsc.VectorSubcoreMesh(core_axis_name="c", subcore_axis_name="s")


def _lanes():
  return lax.broadcasted_iota(jnp.int32, (L,), 0)


def _perm(x, src):
  dn = lax.GatherDimensionNumbers(
      offset_dims=(), collapsed_slice_dims=(0,), start_index_map=(0,))
  return lax.gather(x, src[:, None], dn, (1,),
                    mode=lax.GatherScatterMode.PROMISE_IN_BOUNDS)


def _seg_max(keys, vals, lanes):
  x = vals
  for s in (1, 2, 4, 8):
    src = jnp.maximum(lanes - s, 0)
    pk = _perm(keys, src)
    px = _perm(x, src)
    m = (pk == keys) & (lanes >= s)
    x = jnp.where(m, jnp.maximum(x, px), x)
  return x


def _last_of_seg(keys, lanes):
  nxt = _perm(keys, jnp.minimum(lanes + 1, L - 1))
  return (lanes == L - 1) | (nxt != keys)


def _copy_body(x_ref, o_ref):
  o_ref[...] = x_ref[...]


def _tc_copy(mem):
  blk = M // 32
  return pl.pallas_call(
      _copy_body,
      out_shape=jax.ShapeDtypeStruct((M, D), jnp.float32),
      grid=(32,),
      in_specs=[pl.BlockSpec((blk, D), lambda i: (i, 0))],
      out_specs=pl.BlockSpec((blk, D), lambda i: (i, 0)),
  )(mem)


def _main_body(mem_h, val_h, ioumem_h, iouval_h, usage_h, idx_h,
               nrow_h, pos_h, niou_h, nusage_h,
               idxv, oldb, valb, iouo, iouv, usg,
               tab, idxb, fvalb, mrg, onesb, idx128, sh, ush, sem):
  cid = lax.axis_index("c")
  sid = lax.axis_index("s")
  lanes = _lanes()
  zf = jnp.zeros((L,), jnp.float32)
  is_pos = (cid == 0) & (sid < Q)
  is_iou = (cid == 1) & (sid < Q)
  is_rows = sid >= Q
  is_stage = (cid == 0) & (sid == NS - 1)
  is_uscat = (cid == 0) & (sid >= NS - NU)
  is_extra = sid >= NS - NU
  q = sid

  @pl.when(is_stage)
  def _stage_usage():
    pltpu.sync_copy(usage_h, ush)

  plsc.subcore_barrier()

  def _build(getval, rmw):
    neg1 = jnp.full((L,), -1, jnp.int32)
    zero = jnp.zeros((L,), jnp.int32)
    initval = jnp.where(cid == 0, neg1, zero)

    def initb(i, _):
      tab[pl.ds(i * L, L)] = initval
      return 0

    lax.fori_loop(0, M // L, initb, 0, unroll=8)

    for c in range(BQ // CHB):
      off = q * BQ + c * CHB
      pltpu.sync_copy(idx_h.at[pl.ds(off, CHB)], idxb)
      pltpu.sync_copy(iouval_h.at[pl.ds(off, CHB)], fvalb)

      def body(v, _):
        iv = idxb[pl.ds(v * L, L)]
        sk, sv = plsc.sort_key_val(iv, getval(off, v))
        mx = _seg_max(sk, sv, lanes)
        lm = _last_of_seg(sk, lanes)
        if rmw:
          cur = plsc.load_gather(tab, [sk])
          mx = jnp.maximum(cur, mx)
        plsc.store_scatter(tab, [sk], mx, mask=lm)
        return 0

      lax.fori_loop(0, CHB // L, body, 0, unroll=2)
    pltpu.sync_copy(tab, sh.at[q])

  @pl.when(is_pos)
  def _build_pos():
    _build(lambda off, v: (off + v * L) + lanes, rmw=False)

  @pl.when(is_iou)
  def _build_iou():
    _build(lambda off, v: plsc.bitcast(fvalb[pl.ds(v * L, L)], jnp.int32),
           rmw=True)

  @pl.when(is_uscat)
  def _usage_scatter():
    ones = jnp.ones((L,), jnp.int32)

    def ob(i, _):
      onesb[pl.ds(i * L, L)] = ones
      return 0

    lax.fori_loop(0, UCH // L, ob, 0)
    t = sid - (NS - NU)
    nper = B // UCH // NU

    def uc(c, _):
      off = (t * nper + c) * UCH
      pltpu.sync_copy(idx_h.at[pl.ds(off, UCH)], idx128)
      pltpu.sync_copy(onesb, ush.at[idx128], add=True)
      return 0

    lax.fori_loop(0, nper, uc, 0)

  def do_chunk(c):
    b0 = c * CH
    pltpu.sync_copy(idx_h.at[pl.ds(b0, CH)], idxv)
    hs = []
    for cp in (
        pltpu.make_async_copy(mem_h.at[idxv], oldb, sem),
        pltpu.make_async_copy(val_h.at[pl.ds(b0, CH)], valb, sem),
        pltpu.make_async_copy(ioumem_h.at[idxv], iouo, sem),
        pltpu.make_async_copy(iouval_h.at[pl.ds(b0, CH)], iouv, sem),
        pltpu.make_async_copy(usage_h.at[idxv], usg, sem),
    ):
      cp.start()
      hs.append(cp)
    for h in hs:
      h.wait()

    rws = [g * L + lanes for g in range(NG)]

    def ph1(j, carry):
      col = (j & ~(L - 1)) + ((j + lanes) & (L - 1))
      out = []
      for g in range(NG):
        ov = plsc.load_gather(oldb, [rws[g], col])
        vv = plsc.load_gather(valb, [rws[g], col])
        d, o2, v2 = carry[3 * g:3 * g + 3]
        out += [d + ov * vv, o2 + ov * ov, v2 + vv * vv]
      return tuple(out)

    accs = lax.fori_loop(0, D, ph1, (zf,) * (3 * NG), unroll=4)

    gates, alphas = [], []
    for g in range(NG):
      d, o2, v2 = accs[3 * g:3 * g + 3]
      iouo_g = iouo[pl.ds(g * L, L)]
      iouv_g = iouv[pl.ds(g * L, L)]
      usg_g = usg[pl.ds(g * L, L)].astype(jnp.float32)
      diff = iouv_g - iouo_g + 0.1
      sg = jnp.where(
          diff >= 0.0,
          1.0 / (1.0 + jnp.exp(-diff)),
          jnp.exp(diff) / (1.0 + jnp.exp(diff)),
      )
      uf = 1.0 / (1.0 + usg_g)
      alphas.append(jnp.clip(sg * (0.5 + 0.5 * uf), 0.1, 0.9))
      gates.append((d > 0.0) & (d * d > T2 * (o2 * v2)))

    any_gate = gates[0]
    for g in range(1, NG):
      any_gate = any_gate | gates[g]

    @pl.when(jnp.any(any_gate))
    def _blend():
      def ph2(j, _):
        col = (j & ~(L - 1)) + ((j + lanes) & (L - 1))
        for g in range(NG):
          ov = plsc.load_gather(oldb, [rws[g], col])
          vv = plsc.load_gather(valb, [rws[g], col])
          bl = alphas[g] * vv + (1.0 - alphas[g]) * ov
          plsc.store_scatter(valb, [rws[g], col], jnp.where(gates[g], bl, vv))
        return 0

      lax.fori_loop(0, D, ph2, 0, unroll=2)

    pltpu.sync_copy(valb, nrow_h.at[pl.ds(b0, CH)])

  @pl.when(is_rows)
  def _rows_base():
    k = (sid - Q) * NC + cid
    for i in range(BASE_CHUNKS):
      do_chunk(k * BASE_CHUNKS + i)

  @pl.when(is_extra)
  def _rows_extra():
    do_chunk(NROWT * BASE_CHUNKS + (sid - (NS - NU)) * NC + cid)

  plsc.subcore_barrier()

  @pl.when(is_pos | is_iou)
  def _merge():
    for c in range(M // Q // CHB):
      base = q * (M // Q) + c * CHB
      pltpu.sync_copy(sh.at[0, pl.ds(base, CHB)], mrg)
      for p in range(1, Q):
        pltpu.sync_copy(sh.at[p, pl.ds(base, CHB)], idxb)

        def mx_body(v, _):
          mrg[pl.ds(v * L, L)] = jnp.maximum(
              mrg[pl.ds(v * L, L)], idxb[pl.ds(v * L, L)])
          return 0

        lax.fori_loop(0, CHB // L, mx_body, 0, unroll=4)

      @pl.when(cid == 0)
      def _wpos():
        pltpu.sync_copy(mrg, pos_h.at[pl.ds(base, CHB)])

      @pl.when(cid == 1)
      def _wiou():
        pltpu.sync_copy(ioumem_h.at[pl.ds(base, CHB)], fvalb)

        def cb_body(v, _):
          t2 = plsc.bitcast(mrg[pl.ds(v * L, L)], jnp.float32)
          fvalb[pl.ds(v * L, L)] = jnp.maximum(fvalb[pl.ds(v * L, L)], t2)
          return 0

        lax.fori_loop(0, CHB // L, cb_body, 0, unroll=4)
        pltpu.sync_copy(fvalb, niou_h.at[pl.ds(base, CHB)])

  @pl.when(is_stage)
  def _usage_out():
    pltpu.sync_copy(ush, nusage_h)


def _main(mem, val, iou_mem, iou_val, usage, idx):
  f = pl.kernel(
      _main_body,
      out_type=(
          jax.ShapeDtypeStruct((B, D), jnp.float32),
          jax.ShapeDtypeStruct((M,), jnp.int32),
          jax.ShapeDtypeStruct((M,), jnp.float32),
          jax.ShapeDtypeStruct((M,), jnp.int32),
      ),
      mesh=_mesh(),
      scratch_types=(
          pltpu.VMEM((CH,), jnp.int32),
          pltpu.VMEM((CH, D), jnp.float32),
          pltpu.VMEM((CH, D), jnp.float32),
          pltpu.VMEM((CH,), jnp.float32),
          pltpu.VMEM((CH,), jnp.float32),
          pltpu.VMEM((CH,), jnp.int32),
          pltpu.VMEM((M,), jnp.int32),
          pltpu.VMEM((CHB,), jnp.int32),
          pltpu.VMEM((CHB,), jnp.float32),
          pltpu.VMEM((CHB,), jnp.int32),
          pltpu.VMEM((UCH,), jnp.int32),
          pltpu.VMEM((UCH,), jnp.int32),
          pltpu.VMEM_SHARED((Q, M), jnp.int32),
          pltpu.VMEM_SHARED((M,), jnp.int32),
          pltpu.SemaphoreType.DMA,
      ),
      compiler_params=pltpu.CompilerParams(needs_layout_passes=False),
  )
  return f(mem, val, iou_mem, iou_val, usage, idx)


def _fin_body(nm_in, idx_h, pos_h, nrow_h,
              nm_out,
              idxv, posv, rows, sem):
  del nm_in
  cid = lax.axis_index("c")
  sid = lax.axis_index("s")
  wid = sid * NC + cid
  nb = B // (NC * NS)
  for c in range(nb // CHD):
    b0 = wid * nb + c * CHD
    pltpu.sync_copy(idx_h.at[pl.ds(b0, CHD)], idxv)
    cp_p = pltpu.make_async_copy(pos_h.at[idxv], posv, sem)
    cp_p.start()
    cp_p.wait()
    cp_r = pltpu.make_async_copy(nrow_h.at[posv], rows, sem)
    cp_r.start()
    cp_r.wait()
    cp_s = pltpu.make_async_copy(rows, nm_out.at[idxv], sem)
    cp_s.start()
    cp_s.wait()


def _finalize(nm0, idx, pos, nrow):
  f = _mpmd._mpmd_map(
      [(_mesh(), _fin_body)],
      (jax.ShapeDtypeStruct((M, D), jnp.float32),),
      input_output_aliases={0: 0},
      scratch_types=(
          pltpu.VMEM((CHD,), jnp.int32),
          pltpu.VMEM((CHD,), jnp.int32),
          pltpu.VMEM((CHD, D), jnp.float32),
          pltpu.SemaphoreType.DMA,
      ),
      compiler_params=pltpu.CompilerParams(needs_layout_passes=False),
  )
  (nm,) = f(nm0, idx, pos, nrow)
  return nm


def kernel(mem, val, iou_mem, iou_val, usage, idx):
  idx = idx.astype(jnp.int32)
  nrow, pos, niou, nusage = _main(mem, val, iou_mem, iou_val, usage, idx)
  nm0 = _tc_copy(mem)
  nm = _finalize(nm0, idx, pos, nrow)
  return nm, niou, nusage


# speedup vs baseline: 1.1075x; 1.1075x over previous
import jax
import jax.numpy as jnp
from jax import lax
from jax.experimental import pallas as pl
from jax.experimental.pallas import tpu as pltpu
from jax.experimental.pallas import tpu_sc as plsc
from jax._src.pallas import mpmd as _mpmd

M = 65536
B = 16384
D = 256
NC = 2
NS = 16
L = 16
CH = 64
NCHUNK = B // CH
NROWT = 24
BASE_CHUNKS = NCHUNK // NROWT
CHD = 128
CHB = 2048
Q = 4
BQ = B // Q
UCH = 128
NU = 8
T2 = 0.85 * 0.85
NG = CH // L

_mesh = lambda: plsc.VectorSubcoreMesh(core_axis_name="c", subcore_axis_name="s")


def _lanes():
  return lax.broadcasted_iota(jnp.int32, (L,), 0)


def _perm(x, src):
  dn = lax.GatherDimensionNumbers(
      offset_dims=(), collapsed_slice_dims=(0,), start_index_map=(0,))
  return lax.gather(x, src[:, None], dn, (1,),
                    mode=lax.GatherScatterMode.PROMISE_IN_BOUNDS)


def _seg_max(keys, vals, lanes):
  x = vals
  for s in (1, 2, 4, 8):
    src = jnp.maximum(lanes - s, 0)
    pk = _perm(keys, src)
    px = _perm(x, src)
    m = (pk == keys) & (lanes >= s)
    x = jnp.where(m, jnp.maximum(x, px), x)
  return x


def _last_of_seg(keys, lanes):
  nxt = _perm(keys, jnp.minimum(lanes + 1, L - 1))
  return (lanes == L - 1) | (nxt != keys)


def _copy_body(x_ref, o_ref):
  o_ref[...] = x_ref[...]


def _tc_copy(mem):
  blk = M // 32
  return pl.pallas_call(
      _copy_body,
      out_shape=jax.ShapeDtypeStruct((M, D), jnp.float32),
      grid=(32,),
      in_specs=[pl.BlockSpec((blk, D), lambda i: (i, 0))],
      out_specs=pl.BlockSpec((blk, D), lambda i: (i, 0)),
  )(mem)


def _main_body(mem_h, val_h, ioumem_h, iouval_h, usage_h, idx_h,
               nrow_h, pos_h, niou_h, nusage_h,
               idxb, fvalb, mrg, onesb, idx128, sh, ush):
  cid = lax.axis_index("c")
  sid = lax.axis_index("s")
  lanes = _lanes()
  zf = jnp.zeros((L,), jnp.float32)
  is_pos = (cid == 0) & (sid < Q)
  is_iou = (cid == 1) & (sid < Q)
  is_rows = sid >= Q
  is_stage = (cid == 0) & (sid == NS - 1)
  is_uscat = (cid == 0) & (sid >= NS - NU)
  is_extra = sid >= NS - NU
  q = sid

  @pl.when(is_stage)
  def _stage_usage():
    pltpu.sync_copy(usage_h, ush)

  plsc.subcore_barrier()

  def _build(getval, rmw):
    neg1 = jnp.full((L,), -1, jnp.int32)
    zero = jnp.zeros((L,), jnp.int32)
    initval = jnp.where(cid == 0, neg1, zero)

    def scoped(tab):
      def initb(i, _):
        tab[pl.ds(i * L, L)] = initval
        return 0

      lax.fori_loop(0, M // L, initb, 0, unroll=8)

      for c in range(BQ // CHB):
        off = q * BQ + c * CHB
        pltpu.sync_copy(idx_h.at[pl.ds(off, CHB)], idxb)
        pltpu.sync_copy(iouval_h.at[pl.ds(off, CHB)], fvalb)

        def body(v, _):
          iv = idxb[pl.ds(v * L, L)]
          sk, sv = plsc.sort_key_val(iv, getval(off, v))
          mx = _seg_max(sk, sv, lanes)
          lm = _last_of_seg(sk, lanes)
          if rmw:
            cur = plsc.load_gather(tab, [sk])
            mx = jnp.maximum(cur, mx)
          plsc.store_scatter(tab, [sk], mx, mask=lm)
          return 0

        lax.fori_loop(0, CHB // L, body, 0, unroll=2)
      pltpu.sync_copy(tab, sh.at[q])

    pl.run_scoped(scoped, pltpu.VMEM((M,), jnp.int32))

  @pl.when(is_pos)
  def _build_pos():
    _build(lambda off, v: (off + v * L) + lanes, rmw=False)

  @pl.when(is_iou)
  def _build_iou():
    _build(lambda off, v: plsc.bitcast(fvalb[pl.ds(v * L, L)], jnp.int32),
           rmw=True)

  @pl.when(is_uscat)
  def _usage_scatter():
    ones = jnp.ones((L,), jnp.int32)

    def ob(i, _):
      onesb[pl.ds(i * L, L)] = ones
      return 0

    lax.fori_loop(0, UCH // L, ob, 0)
    t = sid - (NS - NU)
    nper = B // UCH // NU

    def uc(c, _):
      off = (t * nper + c) * UCH
      pltpu.sync_copy(idx_h.at[pl.ds(off, UCH)], idx128)
      pltpu.sync_copy(onesb, ush.at[idx128], add=True)
      return 0

    lax.fori_loop(0, nper, uc, 0)

  def issue_chunk(c, bufset):
    idxv, oldb, valb, iouo, iouv, usg, semi, semo = bufset
    b0 = c * CH
    pltpu.sync_copy(idx_h.at[pl.ds(b0, CH)], idxv)
    hs = []
    for cp in (
        pltpu.make_async_copy(mem_h.at[idxv], oldb, semi),
        pltpu.make_async_copy(val_h.at[pl.ds(b0, CH)], valb, semi),
        pltpu.make_async_copy(ioumem_h.at[idxv], iouo, semi),
        pltpu.make_async_copy(iouval_h.at[pl.ds(b0, CH)], iouv, semi),
        pltpu.make_async_copy(usage_h.at[idxv], usg, semi),
    ):
      cp.start()
      hs.append(cp)
    return hs

  def compute_chunk(c, bufset):
    idxv, oldb, valb, iouo, iouv, usg, semi, semo = bufset
    b0 = c * CH

    rws = [g * L + lanes for g in range(NG)]

    def ph1(j, carry):
      col = (j & ~(L - 1)) + ((j + lanes) & (L - 1))
      out = []
      for g in range(NG):
        ov = plsc.load_gather(oldb, [rws[g], col])
        vv = plsc.load_gather(valb, [rws[g], col])
        d, o2, v2 = carry[3 * g:3 * g + 3]
        out += [d + ov * vv, o2 + ov * ov, v2 + vv * vv]
      return tuple(out)

    accs = lax.fori_loop(0, D, ph1, (zf,) * (3 * NG), unroll=4)

    gates, alphas = [], []
    for g in range(NG):
      d, o2, v2 = accs[3 * g:3 * g + 3]
      iouo_g = iouo[pl.ds(g * L, L)]
      iouv_g = iouv[pl.ds(g * L, L)]
      usg_g = usg[pl.ds(g * L, L)].astype(jnp.float32)
      diff = iouv_g - iouo_g + 0.1
      sg = jnp.where(
          diff >= 0.0,
          1.0 / (1.0 + jnp.exp(-diff)),
          jnp.exp(diff) / (1.0 + jnp.exp(diff)),
      )
      uf = 1.0 / (1.0 + usg_g)
      alphas.append(jnp.clip(sg * (0.5 + 0.5 * uf), 0.1, 0.9))
      gates.append((d > 0.0) & (d * d > T2 * (o2 * v2)))

    any_gate = gates[0]
    for g in range(1, NG):
      any_gate = any_gate | gates[g]

    @pl.when(jnp.any(any_gate))
    def _blend():
      def ph2(j, _):
        col = (j & ~(L - 1)) + ((j + lanes) & (L - 1))
        for g in range(NG):
          ov = plsc.load_gather(oldb, [rws[g], col])
          vv = plsc.load_gather(valb, [rws[g], col])
          bl = alphas[g] * vv + (1.0 - alphas[g]) * ov
          plsc.store_scatter(valb, [rws[g], col], jnp.where(gates[g], bl, vv))
        return 0

      lax.fori_loop(0, D, ph2, 0, unroll=2)

    out_cp = pltpu.make_async_copy(valb, nrow_h.at[pl.ds(b0, CH)], semo)
    out_cp.start()
    return out_cp

  @pl.when(is_rows)
  def _rows_all():
    k = (sid - Q) * NC + cid

    def scoped(*bufs):
      sets = (bufs[0:8], bufs[8:16])
      cids = [k * BASE_CHUNKS + i for i in range(BASE_CHUNKS)]
      pend_in = {0: issue_chunk(cids[0], sets[0])}
      pend_out = {}
      n = len(cids)
      for i in range(n):
        if i + 1 < n:
          if i - 1 in pend_out:
            pend_out.pop(i - 1).wait()
          pend_in[i + 1] = issue_chunk(cids[i + 1], sets[(i + 1) % 2])
        for h in pend_in.pop(i):
          h.wait()
        pend_out[i] = compute_chunk(cids[i], sets[i % 2])
      for i in sorted(pend_out):
        pend_out[i].wait()

      @pl.when(is_extra)
      def _rows_extra():
        e = NROWT * BASE_CHUNKS + (sid - (NS - NU)) * NC + cid
        for h in issue_chunk(e, sets[0]):
          h.wait()
        compute_chunk(e, sets[0]).wait()

    bufset_types = (
        pltpu.VMEM((CH,), jnp.int32),
        pltpu.VMEM((CH, D), jnp.float32),
        pltpu.VMEM((CH, D), jnp.float32),
        pltpu.VMEM((CH,), jnp.float32),
        pltpu.VMEM((CH,), jnp.float32),
        pltpu.VMEM((CH,), jnp.int32),
        pltpu.SemaphoreType.DMA,
        pltpu.SemaphoreType.DMA,
    )
    pl.run_scoped(scoped, *(bufset_types + bufset_types))

  plsc.subcore_barrier()

  @pl.when(is_pos | is_iou)
  def _merge():
    for c in range(M // Q // CHB):
      base = q * (M // Q) + c * CHB
      pltpu.sync_copy(sh.at[0, pl.ds(base, CHB)], mrg)
      for p in range(1, Q):
        pltpu.sync_copy(sh.at[p, pl.ds(base, CHB)], idxb)

        def mx_body(v, _):
          mrg[pl.ds(v * L, L)] = jnp.maximum(
              mrg[pl.ds(v * L, L)], idxb[pl.ds(v * L, L)])
          return 0

        lax.fori_loop(0, CHB // L, mx_body, 0, unroll=4)

      @pl.when(cid == 0)
      def _wpos():
        pltpu.sync_copy(mrg, pos_h.at[pl.ds(base, CHB)])

      @pl.when(cid == 1)
      def _wiou():
        pltpu.sync_copy(ioumem_h.at[pl.ds(base, CHB)], fvalb)

        def cb_body(v, _):
          t2 = plsc.bitcast(mrg[pl.ds(v * L, L)], jnp.float32)
          fvalb[pl.ds(v * L, L)] = jnp.maximum(fvalb[pl.ds(v * L, L)], t2)
          return 0

        lax.fori_loop(0, CHB // L, cb_body, 0, unroll=4)
        pltpu.sync_copy(fvalb, niou_h.at[pl.ds(base, CHB)])

  @pl.when(is_stage)
  def _usage_out():
    pltpu.sync_copy(ush, nusage_h)


def _main(mem, val, iou_mem, iou_val, usage, idx):
  f = pl.kernel(
      _main_body,
      out_type=(
          jax.ShapeDtypeStruct((B, D), jnp.float32),
          jax.ShapeDtypeStruct((M,), jnp.int32),
          jax.ShapeDtypeStruct((M,), jnp.float32),
          jax.ShapeDtypeStruct((M,), jnp.int32),
      ),
      mesh=_mesh(),
      scratch_types=(
          pltpu.VMEM((CHB,), jnp.int32),
          pltpu.VMEM((CHB,), jnp.float32),
          pltpu.VMEM((CHB,), jnp.int32),
          pltpu.VMEM((UCH,), jnp.int32),
          pltpu.VMEM((UCH,), jnp.int32),
          pltpu.VMEM_SHARED((Q, M), jnp.int32),
          pltpu.VMEM_SHARED((M,), jnp.int32),
      ),
      compiler_params=pltpu.CompilerParams(needs_layout_passes=False),
  )
  return f(mem, val, iou_mem, iou_val, usage, idx)


def _fin_body(nm_in, idx_h, pos_h, nrow_h,
              nm_out,
              idxv, posv, rows, sem):
  del nm_in
  cid = lax.axis_index("c")
  sid = lax.axis_index("s")
  wid = sid * NC + cid
  nb = B // (NC * NS)
  for c in range(nb // CHD):
    b0 = wid * nb + c * CHD
    pltpu.sync_copy(idx_h.at[pl.ds(b0, CHD)], idxv)
    cp_p = pltpu.make_async_copy(pos_h.at[idxv], posv, sem)
    cp_p.start()
    cp_p.wait()
    cp_r = pltpu.make_async_copy(nrow_h.at[posv], rows, sem)
    cp_r.start()
    cp_r.wait()
    cp_s = pltpu.make_async_copy(rows, nm_out.at[idxv], sem)
    cp_s.start()
    cp_s.wait()


def _finalize(nm0, idx, pos, nrow):
  f = _mpmd._mpmd_map(
      [(_mesh(), _fin_body)],
      (jax.ShapeDtypeStruct((M, D), jnp.float32),),
      input_output_aliases={0: 0},
      scratch_types=(
          pltpu.VMEM((CHD,), jnp.int32),
          pltpu.VMEM((CHD,), jnp.int32),
          pltpu.VMEM((CHD, D), jnp.float32),
          pltpu.SemaphoreType.DMA,
      ),
      compiler_params=pltpu.CompilerParams(needs_layout_passes=False),
  )
  (nm,) = f(nm0, idx, pos, nrow)
  return nm


def kernel(mem, val, iou_mem, iou_val, usage, idx):
  idx = idx.astype(jnp.int32)
  nrow, pos, niou, nusage = _main(mem, val, iou_mem, iou_val, usage, idx)
  nm0 = _tc_copy(mem)
  nm = _finalize(nm0, idx, pos, nrow)
  return nm, niou, nusage


# speedup vs baseline: 1.2378x; 1.1176x over previous
import jax
import jax.numpy as jnp
from jax import lax
from jax.experimental import pallas as pl
from jax.experimental.pallas import tpu as pltpu
from jax.experimental.pallas import tpu_sc as plsc
from jax._src.pallas import mpmd as _mpmd

M = 65536
B = 16384
D = 256
NC = 2
NS = 16
L = 16
CH = 64
NCHUNK = B // CH
NROWT = 24
BASE_CHUNKS = NCHUNK // NROWT
CHD = 128
CHB = 2048
Q = 4
BQ = B // Q
UCH = 128
NU = 8
T2 = 0.85 * 0.85
NG = CH // L

_mesh = lambda: plsc.VectorSubcoreMesh(core_axis_name="c", subcore_axis_name="s")


def _lanes():
  return lax.broadcasted_iota(jnp.int32, (L,), 0)


def _perm(x, src):
  dn = lax.GatherDimensionNumbers(
      offset_dims=(), collapsed_slice_dims=(0,), start_index_map=(0,))
  return lax.gather(x, src[:, None], dn, (1,),
                    mode=lax.GatherScatterMode.PROMISE_IN_BOUNDS)


def _seg_max(keys, vals, lanes):
  x = vals
  for s in (1, 2, 4, 8):
    src = jnp.maximum(lanes - s, 0)
    pk = _perm(keys, src)
    px = _perm(x, src)
    m = (pk == keys) & (lanes >= s)
    x = jnp.where(m, jnp.maximum(x, px), x)
  return x


def _last_of_seg(keys, lanes):
  nxt = _perm(keys, jnp.minimum(lanes + 1, L - 1))
  return (lanes == L - 1) | (nxt != keys)


def _copy_body(x_ref, o_ref):
  o_ref[...] = x_ref[...]


def _tc_copy(mem):
  blk = M // 32
  return pl.pallas_call(
      _copy_body,
      out_shape=jax.ShapeDtypeStruct((M, D), jnp.float32),
      grid=(32,),
      in_specs=[pl.BlockSpec((blk, D), lambda i: (i, 0))],
      out_specs=pl.BlockSpec((blk, D), lambda i: (i, 0)),
  )(mem)


def _main_body(mem_h, val_h, ioumem_h, iouval_h, usage_h, idx_h,
               nrow_h, p0_h, p1_h, p2_h, p3_h, i0_h, i1_h, i2_h, i3_h,
               idxb, fvalb):
  cid = lax.axis_index("c")
  sid = lax.axis_index("s")
  lanes = _lanes()
  zf = jnp.zeros((L,), jnp.float32)
  is_pos = (cid == 0) & (sid < Q)
  is_iou = (cid == 1) & (sid < Q)
  is_rows = sid >= Q
  is_stage = (cid == 0) & (sid == NS - 1)
  is_uscat = (cid == 0) & (sid >= NS - NU)
  is_extra = sid >= NS - NU
  q = sid

  def _build(getval, rmw, pos):
    neg1 = jnp.full((L,), -1, jnp.int32)
    zero = jnp.zeros((L,), jnp.int32)
    initval = jnp.where(cid == 0, neg1, zero)

    def scoped(tab):
      def initb(i, _):
        tab[pl.ds(i * L, L)] = initval
        return 0

      lax.fori_loop(0, M // L, initb, 0, unroll=8)

      for c in range(BQ // CHB):
        off = q * BQ + c * CHB
        pltpu.sync_copy(idx_h.at[pl.ds(off, CHB)], idxb)
        pltpu.sync_copy(iouval_h.at[pl.ds(off, CHB)], fvalb)

        def body(v, _):
          iv = idxb[pl.ds(v * L, L)]
          sk, sv = plsc.sort_key_val(iv, getval(off, v))
          mx = _seg_max(sk, sv, lanes)
          lm = _last_of_seg(sk, lanes)
          if rmw:
            cur = plsc.load_gather(tab, [sk])
            mx = jnp.maximum(cur, mx)
          plsc.store_scatter(tab, [sk], mx, mask=lm)
          return 0

        lax.fori_loop(0, CHB // L, body, 0, unroll=2)
      outs = (p0_h, p1_h, p2_h, p3_h) if pos else (i0_h, i1_h, i2_h, i3_h)
      for qq in range(Q):
        @pl.when(sid == qq)
        def _pub(qq=qq):
          pltpu.sync_copy(tab, outs[qq])

    pl.run_scoped(scoped, pltpu.VMEM((M,), jnp.int32))

  @pl.when(is_pos)
  def _build_pos():
    _build(lambda off, v: (off + v * L) + lanes, rmw=False, pos=True)

  @pl.when(is_iou)
  def _build_iou():
    _build(lambda off, v: plsc.bitcast(fvalb[pl.ds(v * L, L)], jnp.int32),
           rmw=True, pos=False)

  def issue_chunk(c, bufset):
    idxv, oldb, valb, iouo, iouv, usg, semi, semo = bufset
    b0 = c * CH
    pltpu.sync_copy(idx_h.at[pl.ds(b0, CH)], idxv)
    hs = []
    for cp in (
        pltpu.make_async_copy(mem_h.at[idxv], oldb, semi),
        pltpu.make_async_copy(val_h.at[pl.ds(b0, CH)], valb, semi),
        pltpu.make_async_copy(ioumem_h.at[idxv], iouo, semi),
        pltpu.make_async_copy(iouval_h.at[pl.ds(b0, CH)], iouv, semi),
        pltpu.make_async_copy(usage_h.at[idxv], usg, semi),
    ):
      cp.start()
      hs.append(cp)
    return hs

  def compute_chunk(c, bufset):
    idxv, oldb, valb, iouo, iouv, usg, semi, semo = bufset
    b0 = c * CH

    rws = [g * L + lanes for g in range(NG)]

    def ph1(j, carry):
      col = (j & ~(L - 1)) + ((j + lanes) & (L - 1))
      out = []
      for g in range(NG):
        ov = plsc.load_gather(oldb, [rws[g], col])
        vv = plsc.load_gather(valb, [rws[g], col])
        d, o2, v2 = carry[3 * g:3 * g + 3]
        out += [d + ov * vv, o2 + ov * ov, v2 + vv * vv]
      return tuple(out)

    accs = lax.fori_loop(0, D, ph1, (zf,) * (3 * NG), unroll=4)

    gates, alphas = [], []
    for g in range(NG):
      d, o2, v2 = accs[3 * g:3 * g + 3]
      iouo_g = iouo[pl.ds(g * L, L)]
      iouv_g = iouv[pl.ds(g * L, L)]
      usg_g = usg[pl.ds(g * L, L)].astype(jnp.float32)
      diff = iouv_g - iouo_g + 0.1
      sg = jnp.where(
          diff >= 0.0,
          1.0 / (1.0 + jnp.exp(-diff)),
          jnp.exp(diff) / (1.0 + jnp.exp(diff)),
      )
      uf = 1.0 / (1.0 + usg_g)
      alphas.append(jnp.clip(sg * (0.5 + 0.5 * uf), 0.1, 0.9))
      gates.append((d > 0.0) & (d * d > T2 * (o2 * v2)))

    any_gate = gates[0]
    for g in range(1, NG):
      any_gate = any_gate | gates[g]

    @pl.when(jnp.any(any_gate))
    def _blend():
      def ph2(j, _):
        col = (j & ~(L - 1)) + ((j + lanes) & (L - 1))
        for g in range(NG):
          ov = plsc.load_gather(oldb, [rws[g], col])
          vv = plsc.load_gather(valb, [rws[g], col])
          bl = alphas[g] * vv + (1.0 - alphas[g]) * ov
          plsc.store_scatter(valb, [rws[g], col], jnp.where(gates[g], bl, vv))
        return 0

      lax.fori_loop(0, D, ph2, 0, unroll=2)

    out_cp = pltpu.make_async_copy(valb, nrow_h.at[pl.ds(b0, CH)], semo)
    out_cp.start()
    return out_cp

  @pl.when(is_rows)
  def _rows_all():
    k = (sid - Q) * NC + cid

    def scoped(*bufs):
      sets = (bufs[0:8], bufs[8:16])
      cids = [k * BASE_CHUNKS + i for i in range(BASE_CHUNKS)]
      pend_in = {0: issue_chunk(cids[0], sets[0])}
      pend_out = {}
      n = len(cids)
      for i in range(n):
        if i + 1 < n:
          if i - 1 in pend_out:
            pend_out.pop(i - 1).wait()
          pend_in[i + 1] = issue_chunk(cids[i + 1], sets[(i + 1) % 2])
        for h in pend_in.pop(i):
          h.wait()
        pend_out[i] = compute_chunk(cids[i], sets[i % 2])
      for i in sorted(pend_out):
        pend_out[i].wait()

      @pl.when(is_extra)
      def _rows_extra():
        e = NROWT * BASE_CHUNKS + (sid - (NS - NU)) * NC + cid
        for h in issue_chunk(e, sets[0]):
          h.wait()
        compute_chunk(e, sets[0]).wait()

    bufset_types = (
        pltpu.VMEM((CH,), jnp.int32),
        pltpu.VMEM((CH, D), jnp.float32),
        pltpu.VMEM((CH, D), jnp.float32),
        pltpu.VMEM((CH,), jnp.float32),
        pltpu.VMEM((CH,), jnp.float32),
        pltpu.VMEM((CH,), jnp.int32),
        pltpu.SemaphoreType.DMA,
        pltpu.SemaphoreType.DMA,
    )
    pl.run_scoped(scoped, *(bufset_types + bufset_types))


def _main(mem, val, iou_mem, iou_val, usage, idx):
  f = pl.kernel(
      _main_body,
      out_type=(
          jax.ShapeDtypeStruct((B, D), jnp.float32),
      ) + (jax.ShapeDtypeStruct((M,), jnp.int32),) * (2 * Q),
      mesh=_mesh(),
      scratch_types=(
          pltpu.VMEM((CHB,), jnp.int32),
          pltpu.VMEM((CHB,), jnp.float32),
      ),
      compiler_params=pltpu.CompilerParams(needs_layout_passes=False),
  )
  return f(mem, val, iou_mem, iou_val, usage, idx)


def _fin_body(nm_in, idx_h, nrow_h, ioumem_h, usage_h,
              p0_h, p1_h, p2_h, p3_h, i0_h, i1_h, i2_h, i3_h,
              nm_out, niou_h, nusage_h,
              idxv, pg0, pg1, pg2, pg3, posm, rows,
              mi, ibuf, fvalb, onesb, idx128, ush, sem):
  del nm_in
  cid = lax.axis_index("c")
  sid = lax.axis_index("s")
  lanes = _lanes()
  wid = sid * NC + cid
  nw = NC * NS
  nb = B // nw
  is_stage = (cid == 0) & (sid == NS - 1)
  is_uscat = (cid == 0) & (sid >= NS - NU)

  @pl.when(is_stage)
  def _stage_usage():
    pltpu.sync_copy(usage_h, ush)

  plsc.subcore_barrier()

  @pl.when(is_uscat)
  def _usage_scatter():
    ones = jnp.ones((L,), jnp.int32)

    def ob(i, _):
      onesb[pl.ds(i * L, L)] = ones
      return 0

    lax.fori_loop(0, UCH // L, ob, 0)
    t = sid - (NS - NU)
    nper = B // UCH // NU

    def uc(c, _):
      off = (t * nper + c) * UCH
      pltpu.sync_copy(idx_h.at[pl.ds(off, UCH)], idx128)
      pltpu.sync_copy(onesb, ush.at[idx128], add=True)
      return 0

    lax.fori_loop(0, nper, uc, 0)

  sl = M // nw
  s0 = wid * sl
  pltpu.sync_copy(i0_h.at[pl.ds(s0, sl)], mi)
  for p_h in (i1_h, i2_h, i3_h):
    pltpu.sync_copy(p_h.at[pl.ds(s0, sl)], ibuf)

    def mxb(v, _):
      mi[pl.ds(v * L, L)] = jnp.maximum(mi[pl.ds(v * L, L)],
                                        ibuf[pl.ds(v * L, L)])
      return 0

    lax.fori_loop(0, sl // L, mxb, 0, unroll=4)
  pltpu.sync_copy(ioumem_h.at[pl.ds(s0, sl)], fvalb)

  def cbb(v, _):
    t2 = plsc.bitcast(mi[pl.ds(v * L, L)], jnp.float32)
    fvalb[pl.ds(v * L, L)] = jnp.maximum(fvalb[pl.ds(v * L, L)], t2)
    return 0

  lax.fori_loop(0, sl // L, cbb, 0, unroll=4)
  pltpu.sync_copy(fvalb, niou_h.at[pl.ds(s0, sl)])

  for c in range(nb // CHD):
    b0 = wid * nb + c * CHD
    pltpu.sync_copy(idx_h.at[pl.ds(b0, CHD)], idxv)
    hs = []
    for p_h, pg in ((p0_h, pg0), (p1_h, pg1), (p2_h, pg2), (p3_h, pg3)):
      cp = pltpu.make_async_copy(p_h.at[idxv], pg, sem)
      cp.start()
      hs.append(cp)
    for h in hs:
      h.wait()

    def pm(v, _):
      m01 = jnp.maximum(pg0[pl.ds(v * L, L)], pg1[pl.ds(v * L, L)])
      m23 = jnp.maximum(pg2[pl.ds(v * L, L)], pg3[pl.ds(v * L, L)])
      posm[pl.ds(v * L, L)] = jnp.maximum(m01, m23)
      return 0

    lax.fori_loop(0, CHD // L, pm, 0, unroll=4)
    cp_r = pltpu.make_async_copy(nrow_h.at[posm], rows, sem)
    cp_r.start()
    cp_r.wait()
    cp_s = pltpu.make_async_copy(rows, nm_out.at[idxv], sem)
    cp_s.start()
    cp_s.wait()

  plsc.subcore_barrier()

  @pl.when(is_stage)
  def _usage_out():
    pltpu.sync_copy(ush, nusage_h)


def _finalize(nm0, idx, nrow, iou_mem, usage, parts):
  f = _mpmd._mpmd_map(
      [(_mesh(), _fin_body)],
      (
          jax.ShapeDtypeStruct((M, D), jnp.float32),
          jax.ShapeDtypeStruct((M,), jnp.float32),
          jax.ShapeDtypeStruct((M,), jnp.int32),
      ),
      input_output_aliases={0: 0},
      scratch_types=(
          pltpu.VMEM((CHD,), jnp.int32),
          pltpu.VMEM((CHD,), jnp.int32),
          pltpu.VMEM((CHD,), jnp.int32),
          pltpu.VMEM((CHD,), jnp.int32),
          pltpu.VMEM((CHD,), jnp.int32),
          pltpu.VMEM((CHD,), jnp.int32),
          pltpu.VMEM((CHD, D), jnp.float32),
          pltpu.VMEM((M // (NC * NS),), jnp.int32),
          pltpu.VMEM((M // (NC * NS),), jnp.int32),
          pltpu.VMEM((M // (NC * NS),), jnp.float32),
          pltpu.VMEM((UCH,), jnp.int32),
          pltpu.VMEM((UCH,), jnp.int32),
          pltpu.VMEM_SHARED((M,), jnp.int32),
          pltpu.SemaphoreType.DMA,
      ),
      compiler_params=pltpu.CompilerParams(needs_layout_passes=False),
  )
  return f(nm0, idx, nrow, iou_mem, usage, *parts)


def kernel(mem, val, iou_mem, iou_val, usage, idx):
  idx = idx.astype(jnp.int32)
  out = _main(mem, val, iou_mem, iou_val, usage, idx)
  nrow, parts = out[0], out[1:]
  nm0 = _tc_copy(mem)
  nm, niou, nusage = _finalize(nm0, idx, nrow, iou_mem, usage, parts)
  return nm, niou, nusage


# speedup vs baseline: 1.2789x; 1.0332x over previous
import jax
import jax.numpy as jnp
from jax import lax
from jax.experimental import pallas as pl
from jax.experimental.pallas import tpu as pltpu
from jax.experimental.pallas import tpu_sc as plsc
from jax._src.pallas import mpmd as _mpmd

M = 65536
B = 16384
D = 256
NC = 2
NS = 16
L = 16
CH = 64
NCHUNK = B // CH
NROWT = 24
BASE_CHUNKS = NCHUNK // NROWT
CHD = 128
CHB = 2048
Q = 4
BQ = B // Q
UCH = 128
NU = 8
T2 = 0.85 * 0.85
NG = CH // L

_mesh = lambda: plsc.VectorSubcoreMesh(core_axis_name="c", subcore_axis_name="s")


def _lanes():
  return lax.broadcasted_iota(jnp.int32, (L,), 0)


def _perm(x, src):
  dn = lax.GatherDimensionNumbers(
      offset_dims=(), collapsed_slice_dims=(0,), start_index_map=(0,))
  return lax.gather(x, src[:, None], dn, (1,),
                    mode=lax.GatherScatterMode.PROMISE_IN_BOUNDS)


def _seg_max(keys, vals, lanes):
  x = vals
  for s in (1, 2, 4, 8):
    src = jnp.maximum(lanes - s, 0)
    pk = _perm(keys, src)
    px = _perm(x, src)
    m = (pk == keys) & (lanes >= s)
    x = jnp.where(m, jnp.maximum(x, px), x)
  return x


def _last_of_seg(keys, lanes):
  nxt = _perm(keys, jnp.minimum(lanes + 1, L - 1))
  return (lanes == L - 1) | (nxt != keys)


def _copy_body(x_ref, o_ref):
  o_ref[...] = x_ref[...]


def _tc_copy(mem):
  blk = M // 32
  return pl.pallas_call(
      _copy_body,
      out_shape=jax.ShapeDtypeStruct((M, D), jnp.float32),
      grid=(32,),
      in_specs=[pl.BlockSpec((blk, D), lambda i: (i, 0))],
      out_specs=pl.BlockSpec((blk, D), lambda i: (i, 0)),
  )(mem)


def _main_body(mem_h, val_h, ioumem_h, iouval_h, usage_h, idx_h,
               nrow_h, p0_h, p1_h, p2_h, p3_h, i0_h, i1_h, i2_h, i3_h,
               idxb, fvalb):
  cid = lax.axis_index("c")
  sid = lax.axis_index("s")
  lanes = _lanes()
  zf = jnp.zeros((L,), jnp.float32)
  is_pos = (cid == 0) & (sid < Q)
  is_iou = (cid == 1) & (sid < Q)
  is_rows = sid >= Q
  is_stage = (cid == 0) & (sid == NS - 1)
  is_uscat = (cid == 0) & (sid >= NS - NU)
  is_extra = sid >= NS - NU
  q = sid

  def _build(getval, rmw, pos):
    neg1 = jnp.full((L,), -1, jnp.int32)
    zero = jnp.zeros((L,), jnp.int32)
    initval = jnp.where(cid == 0, neg1, zero)

    def scoped(tab):
      def initb(i, _):
        tab[pl.ds(i * L, L)] = initval
        return 0

      lax.fori_loop(0, M // L, initb, 0, unroll=8)

      for c in range(BQ // CHB):
        off = q * BQ + c * CHB
        pltpu.sync_copy(idx_h.at[pl.ds(off, CHB)], idxb)
        pltpu.sync_copy(iouval_h.at[pl.ds(off, CHB)], fvalb)

        def body(v, _):
          iv = idxb[pl.ds(v * L, L)]
          sk, sv = plsc.sort_key_val(iv, getval(off, v))
          mx = _seg_max(sk, sv, lanes)
          lm = _last_of_seg(sk, lanes)
          if rmw:
            cur = plsc.load_gather(tab, [sk])
            mx = jnp.maximum(cur, mx)
          plsc.store_scatter(tab, [sk], mx, mask=lm)
          return 0

        lax.fori_loop(0, CHB // L, body, 0, unroll=2)
      outs = (p0_h, p1_h, p2_h, p3_h) if pos else (i0_h, i1_h, i2_h, i3_h)
      for qq in range(Q):
        @pl.when(sid == qq)
        def _pub(qq=qq):
          pltpu.sync_copy(tab, outs[qq])

    pl.run_scoped(scoped, pltpu.VMEM((M,), jnp.int32))

  @pl.when(is_pos)
  def _build_pos():
    _build(lambda off, v: (off + v * L) + lanes, rmw=False, pos=True)

  @pl.when(is_iou)
  def _build_iou():
    _build(lambda off, v: plsc.bitcast(fvalb[pl.ds(v * L, L)], jnp.int32),
           rmw=True, pos=False)

  def issue_chunk(c, bufset):
    idxv, oldb, valb, iouo, iouv, usg, semi, semo = bufset
    b0 = c * CH
    pltpu.sync_copy(idx_h.at[pl.ds(b0, CH)], idxv)
    hs = []
    for cp in (
        pltpu.make_async_copy(mem_h.at[idxv], oldb, semi),
        pltpu.make_async_copy(val_h.at[pl.ds(b0, CH)], valb, semi),
        pltpu.make_async_copy(ioumem_h.at[idxv], iouo, semi),
        pltpu.make_async_copy(iouval_h.at[pl.ds(b0, CH)], iouv, semi),
        pltpu.make_async_copy(usage_h.at[idxv], usg, semi),
    ):
      cp.start()
      hs.append(cp)
    return hs

  def compute_chunk(c, bufset):
    idxv, oldb, valb, iouo, iouv, usg, semi, semo = bufset
    b0 = c * CH

    rws = [g * L + lanes for g in range(NG)]

    def ph1(j, carry):
      col = (j & ~(L - 1)) + ((j + lanes) & (L - 1))
      out = []
      for g in range(NG):
        ov = plsc.load_gather(oldb, [rws[g], col])
        vv = plsc.load_gather(valb, [rws[g], col])
        d, o2, v2 = carry[3 * g:3 * g + 3]
        out += [d + ov * vv, o2 + ov * ov, v2 + vv * vv]
      return tuple(out)

    accs = lax.fori_loop(0, D, ph1, (zf,) * (3 * NG), unroll=4)

    gates, alphas = [], []
    for g in range(NG):
      d, o2, v2 = accs[3 * g:3 * g + 3]
      iouo_g = iouo[pl.ds(g * L, L)]
      iouv_g = iouv[pl.ds(g * L, L)]
      usg_g = usg[pl.ds(g * L, L)].astype(jnp.float32)
      diff = iouv_g - iouo_g + 0.1
      sg = jnp.where(
          diff >= 0.0,
          1.0 / (1.0 + jnp.exp(-diff)),
          jnp.exp(diff) / (1.0 + jnp.exp(diff)),
      )
      uf = 1.0 / (1.0 + usg_g)
      alphas.append(jnp.clip(sg * (0.5 + 0.5 * uf), 0.1, 0.9))
      gates.append((d > 0.0) & (d * d > T2 * (o2 * v2)))

    any_gate = gates[0]
    for g in range(1, NG):
      any_gate = any_gate | gates[g]

    @pl.when(jnp.any(any_gate))
    def _blend():
      def ph2(j, _):
        col = (j & ~(L - 1)) + ((j + lanes) & (L - 1))
        for g in range(NG):
          ov = plsc.load_gather(oldb, [rws[g], col])
          vv = plsc.load_gather(valb, [rws[g], col])
          bl = alphas[g] * vv + (1.0 - alphas[g]) * ov
          plsc.store_scatter(valb, [rws[g], col], jnp.where(gates[g], bl, vv))
        return 0

      lax.fori_loop(0, D, ph2, 0, unroll=2)

    out_cp = pltpu.make_async_copy(valb, nrow_h.at[pl.ds(b0, CH)], semo)
    out_cp.start()
    return out_cp

  @pl.when(is_rows)
  def _rows_all():
    k = (sid - Q) * NC + cid

    def scoped(*bufs):
      sets = (bufs[0:8], bufs[8:16])
      cids = [k * BASE_CHUNKS + i for i in range(BASE_CHUNKS)]
      pend_in = {0: issue_chunk(cids[0], sets[0])}
      pend_out = {}
      n = len(cids)
      for i in range(n):
        if i + 1 < n:
          if i - 1 in pend_out:
            pend_out.pop(i - 1).wait()
          pend_in[i + 1] = issue_chunk(cids[i + 1], sets[(i + 1) % 2])
        for h in pend_in.pop(i):
          h.wait()
        pend_out[i] = compute_chunk(cids[i], sets[i % 2])
      for i in sorted(pend_out):
        pend_out[i].wait()

      @pl.when(is_extra)
      def _rows_extra():
        e = NROWT * BASE_CHUNKS + (sid - (NS - NU)) * NC + cid
        for h in issue_chunk(e, sets[0]):
          h.wait()
        compute_chunk(e, sets[0]).wait()

    bufset_types = (
        pltpu.VMEM((CH,), jnp.int32),
        pltpu.VMEM((CH, D), jnp.float32),
        pltpu.VMEM((CH, D), jnp.float32),
        pltpu.VMEM((CH,), jnp.float32),
        pltpu.VMEM((CH,), jnp.float32),
        pltpu.VMEM((CH,), jnp.int32),
        pltpu.SemaphoreType.DMA,
        pltpu.SemaphoreType.DMA,
    )
    pl.run_scoped(scoped, *(bufset_types + bufset_types))


def _main(mem, val, iou_mem, iou_val, usage, idx):
  f = pl.kernel(
      _main_body,
      out_type=(
          jax.ShapeDtypeStruct((B, D), jnp.float32),
      ) + (jax.ShapeDtypeStruct((M,), jnp.int32),) * (2 * Q),
      mesh=_mesh(),
      scratch_types=(
          pltpu.VMEM((CHB,), jnp.int32),
          pltpu.VMEM((CHB,), jnp.float32),
      ),
      compiler_params=pltpu.CompilerParams(needs_layout_passes=False),
  )
  return f(mem, val, iou_mem, iou_val, usage, idx)


def _fin_body(nm_in, idx_h, nrow_h, ioumem_h, usage_h,
              p0_h, p1_h, p2_h, p3_h, i0_h, i1_h, i2_h, i3_h,
              nm_out, niou_h, nusage_h,
              idxvA, pgA0, pgA1, pgA2, pgA3, posmA, rowsA,
              idxvB, pgB0, pgB1, pgB2, pgB3, posmB, rowsB,
              mi, ibuf, fvalb, onesb, idx128, ush, semA, semB):
  del nm_in
  cid = lax.axis_index("c")
  sid = lax.axis_index("s")
  wid = sid * NC + cid
  nw = NC * NS
  nb = B // nw
  is_stage = (cid == 0) & (sid == NS - 1)
  is_uscat = (cid == 0) & (sid >= NS - NU)

  @pl.when(is_stage)
  def _stage_usage():
    pltpu.sync_copy(usage_h, ush)

  plsc.subcore_barrier()

  @pl.when(is_uscat)
  def _usage_scatter():
    ones = jnp.ones((L,), jnp.int32)

    def ob(i, _):
      onesb[pl.ds(i * L, L)] = ones
      return 0

    lax.fori_loop(0, UCH // L, ob, 0)
    t = sid - (NS - NU)
    nper = B // UCH // NU

    def uc(c, _):
      off = (t * nper + c) * UCH
      pltpu.sync_copy(idx_h.at[pl.ds(off, UCH)], idx128)
      pltpu.sync_copy(onesb, ush.at[idx128], add=True)
      return 0

    lax.fori_loop(0, nper, uc, 0)

  sl = M // nw
  s0 = wid * sl
  pltpu.sync_copy(i0_h.at[pl.ds(s0, sl)], mi)
  for p_h in (i1_h, i2_h, i3_h):
    pltpu.sync_copy(p_h.at[pl.ds(s0, sl)], ibuf)

    def mxb(v, _):
      mi[pl.ds(v * L, L)] = jnp.maximum(mi[pl.ds(v * L, L)],
                                        ibuf[pl.ds(v * L, L)])
      return 0

    lax.fori_loop(0, sl // L, mxb, 0, unroll=4)
  pltpu.sync_copy(ioumem_h.at[pl.ds(s0, sl)], fvalb)

  def cbb(v, _):
    t2 = plsc.bitcast(mi[pl.ds(v * L, L)], jnp.float32)
    fvalb[pl.ds(v * L, L)] = jnp.maximum(fvalb[pl.ds(v * L, L)], t2)
    return 0

  lax.fori_loop(0, sl // L, cbb, 0, unroll=4)
  pltpu.sync_copy(fvalb, niou_h.at[pl.ds(s0, sl)])

  psets = ((idxvA, (pgA0, pgA1, pgA2, pgA3), posmA, rowsA, semA),
           (idxvB, (pgB0, pgB1, pgB2, pgB3), posmB, rowsB, semB))
  pparts = (p0_h, p1_h, p2_h, p3_h)
  nchunks = nb // CHD

  def p_issue(c, s):
    sidxv, spg, _sposm, _srows, ssem = psets[s]
    b0 = wid * nb + c * CHD
    pltpu.sync_copy(idx_h.at[pl.ds(b0, CHD)], sidxv)
    hs = []
    for p in range(Q):
      cp = pltpu.make_async_copy(pparts[p].at[sidxv], spg[p], ssem)
      cp.start()
      hs.append(cp)
    return hs

  pend = {0: p_issue(0, 0)}
  pend_out = {}
  for c in range(nchunks):
    s = c % 2
    sidxv, spg, sposm, srows, ssem = psets[s]
    if c + 1 < nchunks:
      if c - 1 in pend_out:
        pend_out.pop(c - 1).wait()
      pend[c + 1] = p_issue(c + 1, (c + 1) % 2)
    for h in pend.pop(c):
      h.wait()

    def pm(v, _):
      m01 = jnp.maximum(spg[0][pl.ds(v * L, L)], spg[1][pl.ds(v * L, L)])
      m23 = jnp.maximum(spg[2][pl.ds(v * L, L)], spg[3][pl.ds(v * L, L)])
      sposm[pl.ds(v * L, L)] = jnp.maximum(m01, m23)
      return 0

    lax.fori_loop(0, CHD // L, pm, 0, unroll=4)
    cp_r = pltpu.make_async_copy(nrow_h.at[sposm], srows, ssem)
    cp_r.start()
    cp_r.wait()
    cp_s = pltpu.make_async_copy(srows, nm_out.at[sidxv], ssem)
    cp_s.start()
    pend_out[c] = cp_s
  for c in sorted(pend_out):
    pend_out[c].wait()

  plsc.subcore_barrier()

  @pl.when(is_stage)
  def _usage_out():
    pltpu.sync_copy(ush, nusage_h)


def _finalize(nm0, idx, nrow, iou_mem, usage, parts):
  pset = (
      pltpu.VMEM((CHD,), jnp.int32),
      pltpu.VMEM((CHD,), jnp.int32),
      pltpu.VMEM((CHD,), jnp.int32),
      pltpu.VMEM((CHD,), jnp.int32),
      pltpu.VMEM((CHD,), jnp.int32),
      pltpu.VMEM((CHD,), jnp.int32),
      pltpu.VMEM((CHD, D), jnp.float32),
  )
  f = _mpmd._mpmd_map(
      [(_mesh(), _fin_body)],
      (
          jax.ShapeDtypeStruct((M, D), jnp.float32),
          jax.ShapeDtypeStruct((M,), jnp.float32),
          jax.ShapeDtypeStruct((M,), jnp.int32),
      ),
      input_output_aliases={0: 0},
      scratch_types=pset + pset + (
          pltpu.VMEM((M // (NC * NS),), jnp.int32),
          pltpu.VMEM((M // (NC * NS),), jnp.int32),
          pltpu.VMEM((M // (NC * NS),), jnp.float32),
          pltpu.VMEM((UCH,), jnp.int32),
          pltpu.VMEM((UCH,), jnp.int32),
          pltpu.VMEM_SHARED((M,), jnp.int32),
          pltpu.SemaphoreType.DMA,
          pltpu.SemaphoreType.DMA,
      ),
      compiler_params=pltpu.CompilerParams(needs_layout_passes=False),
  )
  return f(nm0, idx, nrow, iou_mem, usage, *parts)


def kernel(mem, val, iou_mem, iou_val, usage, idx):
  idx = idx.astype(jnp.int32)
  out = _main(mem, val, iou_mem, iou_val, usage, idx)
  nrow, parts = out[0], out[1:]
  nm0 = _tc_copy(mem)
  nm, niou, nusage = _finalize(nm0, idx, nrow, iou_mem, usage, parts)
  return nm, niou, nusage
